# baseline (speedup 1.0000x reference)
import functools
import jax
import jax.numpy as jnp
from jax import lax
from jax.experimental import pallas as pl
from jax.experimental.pallas import tpu as pltpu
from jax.experimental.pallas import tpu_sc as plsc

_K = 1024
_D = 64
_N = 32768
_BLK = 1024
_GRID = _N // _BLK
_BETA = 0.25


def _vq_body(x_ref, e_ref, et2_ref, cn_ref, iotac_ref, oh_ref, idx_ref,
             loss_ref, perp_ref, cnt_ref, acc_ref):
    i = pl.program_id(0)
    x = x_ref[...]
    et2 = et2_ref[...]

    @pl.when(i == 0)
    def _():
        cnt_ref[...] = jnp.zeros_like(cnt_ref)
        acc_ref[0] = 0.0

    rn = jnp.sum(x * x, axis=1, keepdims=True)
    mm2 = jnp.dot(x, et2)
    dist = (rn + cn_ref[...]) - mm2

    mn = jnp.min(dist, axis=1, keepdims=True)
    oh = (dist == mn).astype(jnp.float32)
    oh_ref[...] = oh
    acc_ref[0] += jnp.sum(mn)
    idxf = jnp.dot(oh, iotac_ref[...])
    idx_ref[...] = idxf.astype(jnp.int32)
    cnt8 = jnp.dot(jnp.ones((8, _BLK), jnp.float32), oh)
    cnt0 = cnt8[0:1]
    cnt_ref[...] += cnt0
    nsel = jnp.sum(cnt0)

    @pl.when(nsel != jnp.float32(_BLK))
    def _():
        iota = lax.broadcasted_iota(jnp.int32, (_BLK, _K), 1)
        idx2 = jnp.min(jnp.where(dist == mn, iota, _K), axis=1, keepdims=True)
        oh2 = (iota == idx2).astype(jnp.float32)
        oh_ref[...] = oh2
        idx_ref[...] = idx2
        cnt_ref[...] += jnp.sum(oh2, axis=0, keepdims=True) - cnt0

    @pl.when(i == _GRID - 1)
    def _():
        m = acc_ref[0] / jnp.float32(_N * _D)
        loss_ref[...] = jnp.full((1, 1), m * _BETA + m, jnp.float32)
        avg = cnt_ref[...] / jnp.float32(_N)
        ent = jnp.sum(avg * jnp.log(avg + 1e-10))
        perp_ref[...] = jnp.full((1, 1), jnp.exp(-ent), jnp.float32)


def _tc_part(x, embedding, cn, iotac):
    return pl.pallas_call(
        _vq_body,
        grid=(_GRID,),
        in_specs=[
            pl.BlockSpec((_BLK, _D), lambda i: (i, 0)),
            pl.BlockSpec((_K, _D), lambda i: (0, 0)),
            pl.BlockSpec((_D, _K), lambda i: (0, 0)),
            pl.BlockSpec((1, _K), lambda i: (0, 0)),
            pl.BlockSpec((_K, 1), lambda i: (0, 0)),
        ],
        out_specs=[
            pl.BlockSpec((_BLK, _K), lambda i: (i, 0)),
            pl.BlockSpec((_BLK, 1), lambda i: (i, 0)),
            pl.BlockSpec((1, 1), lambda i: (0, 0)),
            pl.BlockSpec((1, 1), lambda i: (0, 0)),
        ],
        out_shape=[
            jax.ShapeDtypeStruct((_N, _K), jnp.float32),
            jax.ShapeDtypeStruct((_N, 1), jnp.int32),
            jax.ShapeDtypeStruct((1, 1), jnp.float32),
            jax.ShapeDtypeStruct((1, 1), jnp.float32),
        ],
        scratch_shapes=[
            pltpu.VMEM((1, _K), jnp.float32),
            pltpu.SMEM((1,), jnp.float32),
        ],
        compiler_params=pltpu.CompilerParams(
            dimension_semantics=("arbitrary",)),
    )(x, embedding, embedding.T * 2.0, cn, iotac)


_DP = 128
_CH = 512


def _make_sc_gather():
    info = plsc.get_sparse_core_info()
    nw = info.num_cores * info.num_subcores
    b_per_w = _N // nw
    mesh = plsc.VectorSubcoreMesh(core_axis_name="c", subcore_axis_name="s")

    @functools.partial(
        pl.kernel, mesh=mesh,
        out_type=jax.ShapeDtypeStruct((_N, _DP), jnp.float32),
        scratch_types=[
            pltpu.VMEM((_CH,), jnp.int32),
            pltpu.VMEM((_CH, _DP), jnp.float32),
            pltpu.SemaphoreType.DMA,
        ],
    )
    def gather(table_hbm, idx_hbm, out_hbm, idx_v, rows_v, sem):
        wid = lax.axis_index("s") * info.num_cores + lax.axis_index("c")
        base = wid * b_per_w
        for j in range(b_per_w // _CH):
            pltpu.sync_copy(idx_hbm.at[pl.ds(base + j * _CH, _CH)], idx_v)
            pltpu.async_copy(table_hbm.at[idx_v], rows_v, sem).wait()
            pltpu.sync_copy(rows_v, out_hbm.at[pl.ds(base + j * _CH, _CH)])

    return gather


def kernel(latents, embedding):
    b, c, h, w = latents.shape
    x = jnp.transpose(latents, (0, 2, 3, 1)).reshape(-1, _D)
    cn = jnp.sum(embedding ** 2, axis=1)[None, :]
    iotac = lax.iota(jnp.float32, _K)[:, None]
    oh, idx, loss, perp = _tc_part(x, embedding, cn, iotac)
    table = jnp.pad(embedding, ((0, 0), (0, _DP - _D)))
    qst_pad = _make_sc_gather()(table, idx.reshape(-1))
    qst4 = jnp.transpose(
        qst_pad.reshape(b, h, w, _DP)[..., :_D], (0, 3, 1, 2))
    return (loss[0, 0], perp[0, 0], qst4, oh)
